# baseline (speedup 1.0000x reference)
import functools

import jax
import jax.numpy as jnp
from jax.experimental import pallas as pl

_CS = [64, 128, 256, 512]
_SHAPES = [(64, 70, 70), (128, 40, 40), (256, 24, 24), (512, 12, 12)]
_KS = [7, 5, 3, 1]
_TOPK = [4, 3, 3, 1]
_HEADS = 4
_T_NUM = 2
_EPS = 1e-5
_PREC = jax.lax.Precision.DEFAULT


def _dot(a, b, dims):
    return jax.lax.dot_general(a, b, (dims, ((), ())), precision=_PREC,
                               preferred_element_type=jnp.float32)


def _ln(x, g, b):
    mu = jnp.mean(x, axis=-1, keepdims=True)
    var = jnp.mean((x - mu) ** 2, axis=-1, keepdims=True)
    return (x - mu) / jnp.sqrt(var + _EPS) * g + b


def _select_body(K, u_ref, out_ref):
    u = u_ref[0]
    L, v, C = u.shape
    ub = u.astype(jnp.bfloat16).astype(jnp.float32)
    s = jnp.sum(ub, axis=1)
    rel = jnp.sum(ub * s[:, None, :], axis=2) / v
    iota = jax.lax.broadcasted_iota(jnp.int32, (L, v), 1)
    picks = []
    for _ in range(K):
        mn = jnp.min(rel, axis=1, keepdims=True)
        cand = jnp.where(rel == mn, iota, v)
        first = jnp.min(cand, axis=1, keepdims=True)
        onehot = iota == first
        w = onehot.astype(jnp.float32)
        picks.append(jnp.sum(u * w[:, :, None], axis=1))
        rel = jnp.where(onehot, jnp.inf, rel)
    sel = jnp.concatenate([p[:, None, :] for p in picks], axis=1)
    out_ref[0] = sel


def _select_tokens(u, K):
    B, L, v, C = u.shape
    out = pl.pallas_call(
        functools.partial(_select_body, K),
        grid=(B,),
        in_specs=[pl.BlockSpec((1, L, v, C), lambda b: (b, 0, 0, 0))],
        out_specs=pl.BlockSpec((1, L, K, C), lambda b: (b, 0, 0, 0)),
        out_shape=jax.ShapeDtypeStruct((B, L, K, C), jnp.float32),
    )(u)
    return out.reshape(B, L * K, C)


def _unfold(x, k):
    B, C, H, W = x.shape
    Hp, Wp = H // k, W // k
    u = x.reshape(B, C, Hp, k, Wp, k)
    return jnp.transpose(u, (0, 2, 4, 3, 5, 1)).reshape(B, Hp * Wp, k * k, C)


def _kvproj_body(inner, x_ref, g_ref, b_ref, w_ref, k_ref, v_ref):
    x = _ln(x_ref[0], g_ref[0], b_ref[0])
    w = w_ref[...]
    kvt = _dot(w, x, ((1,), (1,)))
    k_ref[0] = kvt[:inner, :]
    v_ref[0] = kvt[inner:, :]


def _kv_project(skip, g, b, w, inner):
    B, M, Cj = skip.shape
    kt, vt = pl.pallas_call(
        functools.partial(_kvproj_body, inner),
        grid=(B,),
        in_specs=[
            pl.BlockSpec((1, M, Cj), lambda bb: (bb, 0, 0)),
            pl.BlockSpec((1, Cj), lambda bb: (0, 0)),
            pl.BlockSpec((1, Cj), lambda bb: (0, 0)),
            pl.BlockSpec((2 * inner, Cj), lambda bb: (0, 0)),
        ],
        out_specs=[
            pl.BlockSpec((1, inner, M), lambda bb: (bb, 0, 0)),
            pl.BlockSpec((1, inner, M), lambda bb: (bb, 0, 0)),
        ],
        out_shape=[
            jax.ShapeDtypeStruct((B, inner, M), jnp.float32),
            jax.ShapeDtypeStruct((B, inner, M), jnp.float32),
        ],
    )(skip, g.reshape(1, Cj), b.reshape(1, Cj), w)
    return kt, vt


def _pad128(m):
    return -(-m // 128) * 128


def _ln_cols(xt, g, b):
    C = xt.shape[0]
    ones = jnp.ones((1, C), jnp.float32)
    s1 = _dot(ones, xt, ((1,), (0,)))
    s2 = _dot(ones, xt * xt, ((1,), (0,)))
    mu = s1 / C
    var = s2 / C - mu * mu
    return (xt - mu) * jax.lax.rsqrt(var + _EPS) * g + b


def _fold_body(dh, Ms, layouts, x_refs, g_refs, b_refs, w_refs, wq_ref,
               wot_ref, kk_ref, vv_ref):
    inner = _HEADS * dh
    S1 = sum(_pad128(m) for m in Ms)
    off = 0
    for c in range(3):
        M = Ms[c]
        Mp = _pad128(M)
        if layouts[c] == 'ch':
            xt = _ln_cols(x_refs[c][0], g_refs[c][...], b_refs[c][...])
            kvt = _dot(w_refs[c][...], xt, ((1,), (0,)))
        else:
            x = _ln(x_refs[c][0], g_refs[c][0], b_refs[c][0])
            kvt = _dot(w_refs[c][...], x, ((1,), (1,)))
        for h in range(_HEADS):
            kh = kvt[h * dh:(h + 1) * dh, :]
            wqh = wq_ref[h * dh:(h + 1) * dh, :]
            kk = _dot(wqh, kh, ((0,), (0,)))
            vh = kvt[inner + h * dh:inner + (h + 1) * dh, :]
            wth = wot_ref[(h * 3 + c) * dh:(h * 3 + c + 1) * dh, :]
            vv = _dot(vh, wth, ((0,), (0,)))
            if Mp != M:
                C = kk.shape[0]
                kk = jnp.concatenate(
                    [kk, jnp.zeros((C, Mp - M), jnp.float32)], axis=1)
                vv = jnp.concatenate(
                    [vv, jnp.zeros((Mp - M, C), jnp.float32)], axis=0)
            kk_ref[0, :, h * S1 + off:h * S1 + off + Mp] = kk
            vv_ref[0, h * S1 + off:h * S1 + off + Mp, :] = vv
        off += Mp


def _fold_kv(skips, layouts, gs, bs, ws, wq, wout, dh):
    B = skips[0].shape[0]
    C = wq.shape[1]
    inner = _HEADS * dh
    Ms = [s.shape[1] if layouts[c] == 'tok' else s.shape[2]
          for c, s in enumerate(skips)]
    S = _HEADS * sum(_pad128(m) for m in Ms)

    def body(s0, g0, b0, w0, s1, g1, b1, w1, s2, g2, b2, w2, wqr, wot,
             kk_ref, vv_ref):
        _fold_body(dh, Ms, layouts, (s0, s1, s2), (g0, g1, g2), (b0, b1, b2),
                   (w0, w1, w2), wqr, wot, kk_ref, vv_ref)

    in_specs = []
    args = []
    for c in range(3):
        Cj = ws[c].shape[1]
        M = Ms[c]
        if layouts[c] == 'ch':
            in_specs.append(pl.BlockSpec((1, Cj, M), lambda bb: (bb, 0, 0)))
            in_specs += [pl.BlockSpec((Cj, 1), lambda bb: (0, 0))] * 2
            args += [skips[c], gs[c].reshape(Cj, 1), bs[c].reshape(Cj, 1)]
        else:
            in_specs.append(pl.BlockSpec((1, M, Cj), lambda bb: (bb, 0, 0)))
            in_specs += [pl.BlockSpec((1, Cj), lambda bb: (0, 0))] * 2
            args += [skips[c], gs[c].reshape(1, Cj), bs[c].reshape(1, Cj)]
        in_specs.append(pl.BlockSpec((2 * inner, Cj), lambda bb: (0, 0)))
        args.append(ws[c])
    in_specs += [
        pl.BlockSpec((inner, C), lambda bb: (0, 0)),
        pl.BlockSpec((3 * inner, C), lambda bb: (0, 0)),
    ]
    args += [wq, wout.T]
    return pl.pallas_call(
        body,
        grid=(B,),
        in_specs=in_specs,
        out_specs=[
            pl.BlockSpec((1, C, S), lambda bb: (bb, 0, 0)),
            pl.BlockSpec((1, S, C), lambda bb: (bb, 0, 0)),
        ],
        out_shape=[
            jax.ShapeDtypeStruct((B, C, S), jnp.float32),
            jax.ShapeDtypeStruct((B, S, C), jnp.float32),
        ],
    )(*args)


def _attn_fold_body(scale, segs, x_ref, g_ref, b_ref, kk_ref, vv_ref,
                    bias_ref, out_ref):
    xn = _ln_cols(x_ref[0], g_ref[...], b_ref[...])
    d = _dot(xn, kk_ref[0], ((0,), (0,))) * scale + bias_ref[...]
    parts = []
    for off, w in segs:
        ds = d[:, off:off + w]
        m = jnp.max(ds, axis=1, keepdims=True)
        e = jnp.exp(ds - m)
        parts.append(e / jnp.sum(e, axis=1, keepdims=True))
    a = jnp.concatenate(parts, axis=1)
    out_ref[0] = _dot(vv_ref[0], a, ((0,), (1,)))


def _attention_fold(x, kk, vv, g, b, Ms, dh, bn):
    B, C, N = x.shape
    S = kk.shape[2]
    S1 = S // _HEADS
    nb = -(-N // bn)
    Np = nb * bn
    if Np != N:
        x = jnp.pad(x, ((0, 0), (0, 0), (0, Np - N)))
    scale = dh ** (-0.5)
    segs = []
    bias = jnp.zeros((S,), jnp.float32)
    for h in range(_HEADS):
        off = 0
        for M in Ms:
            Mp = _pad128(M)
            segs.append((h * S1 + off, Mp))
            if Mp != M:
                bias = bias.at[h * S1 + off + M:h * S1 + off + Mp].set(-1e30)
            off += Mp
    out = pl.pallas_call(
        functools.partial(_attn_fold_body, scale, segs),
        grid=(B, nb),
        in_specs=[
            pl.BlockSpec((1, C, bn), lambda bb, nn: (bb, 0, nn)),
            pl.BlockSpec((C, 1), lambda bb, nn: (0, 0)),
            pl.BlockSpec((C, 1), lambda bb, nn: (0, 0)),
            pl.BlockSpec((1, C, S), lambda bb, nn: (bb, 0, 0)),
            pl.BlockSpec((1, S, C), lambda bb, nn: (bb, 0, 0)),
            pl.BlockSpec((1, S), lambda bb, nn: (0, 0)),
        ],
        out_specs=pl.BlockSpec((1, C, bn), lambda bb, nn: (bb, 0, nn)),
        out_shape=jax.ShapeDtypeStruct((B, C, Np), jnp.float32),
    )(x, g.reshape(C, 1), b.reshape(C, 1), kk, vv, bias.reshape(1, S))
    return out[:, :, :N]


def _attn_body(dh, scale, q_ref, k0, v0, k1, v1, k2, v2,
               g_ref, b_ref, wq_ref, wot_ref, out_ref):
    ks = (k0, k1, k2)
    vs = (v0, v1, v2)
    xn = _ln_cols(q_ref[0], g_ref[...], b_ref[...])
    C, BN = xn.shape
    acc = jnp.zeros((C, BN), jnp.float32)
    for h in range(_HEADS):
        wq_h = wq_ref[h * dh:(h + 1) * dh, :]
        qh = _dot(wq_h, xn, ((1,), (0,)))
        for c in range(3):
            kh = ks[c][0][h * dh:(h + 1) * dh, :]
            d = _dot(qh, kh, ((0,), (0,))) * scale
            d = d - jnp.max(d, axis=1, keepdims=True)
            e = jnp.exp(d)
            a = e / jnp.sum(e, axis=1, keepdims=True)
            vh = vs[c][0][h * dh:(h + 1) * dh, :]
            o = _dot(a, vh, ((1,), (1,)))
            wsl = wot_ref[(h * 3 + c) * dh:(h * 3 + c + 1) * dh, :]
            acc = acc + _dot(wsl, o, ((0,), (1,)))
    out_ref[0] = acc


def _attention(x, kts, vts, g, b, wq, wout, dh, bn):
    B, C, N = x.shape
    inner = _HEADS * dh
    nb = -(-N // bn)
    Np = nb * bn
    if Np != N:
        x = jnp.pad(x, ((0, 0), (0, 0), (0, Np - N)))
    scale = dh ** (-0.5)
    kv_specs = []
    for kt in kts:
        M = kt.shape[2]
        kv_specs.append(pl.BlockSpec((1, inner, M), lambda bb, nn: (bb, 0, 0)))
        kv_specs.append(pl.BlockSpec((1, inner, M), lambda bb, nn: (bb, 0, 0)))
    out = pl.pallas_call(
        functools.partial(_attn_body, dh, scale),
        grid=(B, nb),
        in_specs=[pl.BlockSpec((1, C, bn), lambda bb, nn: (bb, 0, nn))]
        + kv_specs
        + [
            pl.BlockSpec((C, 1), lambda bb, nn: (0, 0)),
            pl.BlockSpec((C, 1), lambda bb, nn: (0, 0)),
            pl.BlockSpec((inner, C), lambda bb, nn: (0, 0)),
            pl.BlockSpec((3 * inner, C), lambda bb, nn: (0, 0)),
        ],
        out_specs=pl.BlockSpec((1, C, bn), lambda bb, nn: (bb, 0, nn)),
        out_shape=jax.ShapeDtypeStruct((B, C, Np), jnp.float32),
    )(x, kts[0], vts[0], kts[1], vts[1], kts[2], vts[2],
      g.reshape(C, 1), b.reshape(C, 1), wq, wout.T)
    return out[:, :, :N]


_BN = [512, 256, 576, 144]


def _smla(xs, blocks):
    B = xs[0].shape[0]
    flat = []
    tmp_skips = []
    for i, x in enumerate(xs):
        C, H, W = _SHAPES[i]
        flat.append(x.reshape(B, C, H * W))
        if _KS[i] == 1:
            tmp_skips.append(flat[i])
        else:
            u = _unfold(x, _KS[i])
            tmp_skips.append(_select_tokens(u, _TOPK[i]))
    outs = []
    for idx in range(4):
        ap = blocks[idx]
        Ci = _CS[idx]
        inner = _HEADS * Ci
        skips = [tmp_skips[j] for j in range(4) if j != idx]
        if idx < 3:
            layouts = ['tok', 'tok', 'ch']
            Ms = [skips[0].shape[1], skips[1].shape[1], skips[2].shape[2]]
            kk, vv = _fold_kv(skips, layouts, ap['kvn_g'], ap['kvn_b'],
                              ap['Wkv'], ap['Wq'], ap['Wout'], Ci)
            o = _attention_fold(flat[idx], kk, vv, ap['qn_g'], ap['qn_b'],
                                Ms, Ci, _BN[idx])
        else:
            kts, vts = [], []
            for c in range(3):
                kt, vt = _kv_project(skips[c], ap['kvn_g'][c],
                                     ap['kvn_b'][c], ap['Wkv'][c], inner)
                kts.append(kt)
                vts.append(vt)
            o = _attention(flat[idx], kts, vts, ap['qn_g'], ap['qn_b'],
                           ap['Wq'], ap['Wout'], Ci, _BN[idx])
        C, H, W = _SHAPES[idx]
        outs.append(o.reshape(B, C, H, W))
    return outs


def kernel(x0, x1, x2, x3, params):
    xs = [x0, x1, x2, x3]
    for t in range(_T_NUM):
        xs = _smla(xs, params[t])
    return tuple(xs)


# speedup vs baseline: 1.1427x; 1.1427x over previous
import functools

import jax
import jax.numpy as jnp
from jax.experimental import pallas as pl

_CS = [64, 128, 256, 512]
_SHAPES = [(64, 70, 70), (128, 40, 40), (256, 24, 24), (512, 12, 12)]
_KS = [7, 5, 3, 1]
_TOPK = [4, 3, 3, 1]
_HEADS = 4
_T_NUM = 2
_EPS = 1e-5
_PREC = jax.lax.Precision.DEFAULT


def _dot(a, b, dims):
    return jax.lax.dot_general(a, b, (dims, ((), ())), precision=_PREC,
                               preferred_element_type=jnp.float32)


def _ln(x, g, b):
    mu = jnp.mean(x, axis=-1, keepdims=True)
    var = jnp.mean((x - mu) ** 2, axis=-1, keepdims=True)
    return (x - mu) / jnp.sqrt(var + _EPS) * g + b


def _select_body(K, u_ref, out_ref):
    u = u_ref[0]
    L, v, C = u.shape
    ub = u.astype(jnp.bfloat16).astype(jnp.float32)
    s = jnp.sum(ub, axis=1)
    rel = jnp.sum(ub * s[:, None, :], axis=2) / v
    iota = jax.lax.broadcasted_iota(jnp.int32, (L, v), 1)
    picks = []
    for _ in range(K):
        mn = jnp.min(rel, axis=1, keepdims=True)
        cand = jnp.where(rel == mn, iota, v)
        first = jnp.min(cand, axis=1, keepdims=True)
        onehot = iota == first
        w = onehot.astype(jnp.float32)
        picks.append(jnp.sum(u * w[:, :, None], axis=1))
        rel = jnp.where(onehot, jnp.inf, rel)
    sel = jnp.concatenate([p[:, None, :] for p in picks], axis=1)
    out_ref[0] = sel


def _select_tokens(u, K):
    B, L, v, C = u.shape
    out = pl.pallas_call(
        functools.partial(_select_body, K),
        grid=(B,),
        in_specs=[pl.BlockSpec((1, L, v, C), lambda b: (b, 0, 0, 0))],
        out_specs=pl.BlockSpec((1, L, K, C), lambda b: (b, 0, 0, 0)),
        out_shape=jax.ShapeDtypeStruct((B, L, K, C), jnp.float32),
    )(u)
    return out.reshape(B, L * K, C)


def _unfold(x, k):
    B, C, H, W = x.shape
    Hp, Wp = H // k, W // k
    u = x.reshape(B, C, Hp, k, Wp, k)
    return jnp.transpose(u, (0, 2, 4, 3, 5, 1)).reshape(B, Hp * Wp, k * k, C)


def _kvproj_body(inner, x_ref, g_ref, b_ref, w_ref, k_ref, v_ref):
    x = _ln(x_ref[0], g_ref[0], b_ref[0])
    w = w_ref[...]
    kvt = _dot(w, x, ((1,), (1,)))
    k_ref[0] = kvt[:inner, :]
    v_ref[0] = kvt[inner:, :]


def _kv_project(skip, g, b, w, inner):
    B, M, Cj = skip.shape
    kt, vt = pl.pallas_call(
        functools.partial(_kvproj_body, inner),
        grid=(B,),
        in_specs=[
            pl.BlockSpec((1, M, Cj), lambda bb: (bb, 0, 0)),
            pl.BlockSpec((1, Cj), lambda bb: (0, 0)),
            pl.BlockSpec((1, Cj), lambda bb: (0, 0)),
            pl.BlockSpec((2 * inner, Cj), lambda bb: (0, 0)),
        ],
        out_specs=[
            pl.BlockSpec((1, inner, M), lambda bb: (bb, 0, 0)),
            pl.BlockSpec((1, inner, M), lambda bb: (bb, 0, 0)),
        ],
        out_shape=[
            jax.ShapeDtypeStruct((B, inner, M), jnp.float32),
            jax.ShapeDtypeStruct((B, inner, M), jnp.float32),
        ],
    )(skip, g.reshape(1, Cj), b.reshape(1, Cj), w)
    return kt, vt


def _pad128(m):
    return -(-m // 128) * 128


def _fold_body(dh, Ms, x_refs, g_refs, b_refs, w_refs, wq_ref, wot_ref,
               kk_ref, vv_ref):
    inner = _HEADS * dh
    S1 = sum(_pad128(m) for m in Ms)
    off = 0
    for c in range(3):
        M = Ms[c]
        Mp = _pad128(M)
        x = _ln(x_refs[c][0], g_refs[c][0], b_refs[c][0])
        kvt = _dot(w_refs[c][...], x, ((1,), (1,)))
        for h in range(_HEADS):
            kh = kvt[h * dh:(h + 1) * dh, :]
            wqh = wq_ref[h * dh:(h + 1) * dh, :]
            kk = _dot(wqh, kh, ((0,), (0,)))
            vh = kvt[inner + h * dh:inner + (h + 1) * dh, :]
            wth = wot_ref[(h * 3 + c) * dh:(h * 3 + c + 1) * dh, :]
            vv = _dot(vh, wth, ((0,), (0,)))
            if Mp != M:
                C = kk.shape[0]
                kk = jnp.concatenate(
                    [kk, jnp.zeros((C, Mp - M), jnp.float32)], axis=1)
                vv = jnp.concatenate(
                    [vv, jnp.zeros((Mp - M, C), jnp.float32)], axis=0)
            kk_ref[0, :, h * S1 + off:h * S1 + off + Mp] = kk
            vv_ref[0, h * S1 + off:h * S1 + off + Mp, :] = vv
        off += Mp


def _fold_kv(skips, gs, bs, ws, wq, wout, dh):
    B = skips[0].shape[0]
    C = wq.shape[1]
    inner = _HEADS * dh
    Ms = [s.shape[1] for s in skips]
    S = _HEADS * sum(_pad128(m) for m in Ms)

    def body(s0, g0, b0, w0, s1, g1, b1, w1, s2, g2, b2, w2, wqr, wot,
             kk_ref, vv_ref):
        _fold_body(dh, Ms, (s0, s1, s2), (g0, g1, g2), (b0, b1, b2),
                   (w0, w1, w2), wqr, wot, kk_ref, vv_ref)

    in_specs = []
    args = []
    for c in range(3):
        M, Cj = skips[c].shape[1], skips[c].shape[2]
        in_specs += [
            pl.BlockSpec((1, M, Cj), lambda bb: (bb, 0, 0)),
            pl.BlockSpec((1, Cj), lambda bb: (0, 0)),
            pl.BlockSpec((1, Cj), lambda bb: (0, 0)),
            pl.BlockSpec((2 * inner, Cj), lambda bb: (0, 0)),
        ]
        args += [skips[c], gs[c].reshape(1, Cj), bs[c].reshape(1, Cj), ws[c]]
    in_specs += [
        pl.BlockSpec((inner, C), lambda bb: (0, 0)),
        pl.BlockSpec((3 * inner, C), lambda bb: (0, 0)),
    ]
    args += [wq, wout.T]
    return pl.pallas_call(
        body,
        grid=(B,),
        in_specs=in_specs,
        out_specs=[
            pl.BlockSpec((1, C, S), lambda bb: (bb, 0, 0)),
            pl.BlockSpec((1, S, C), lambda bb: (bb, 0, 0)),
        ],
        out_shape=[
            jax.ShapeDtypeStruct((B, C, S), jnp.float32),
            jax.ShapeDtypeStruct((B, S, C), jnp.float32),
        ],
    )(*args)


def _attn_fold_body(scale, segs, x_ref, g_ref, b_ref, kk_ref, vv_ref,
                    bias_ref, out_ref):
    x = x_ref[0]
    BN, C = x.shape
    one = jnp.ones((C, 1), jnp.float32)
    s1 = _dot(x, one, ((1,), (0,)))
    s2 = _dot(x * x, one, ((1,), (0,)))
    mu = s1 / C
    var = s2 / C - mu * mu
    xn = (x - mu) * jax.lax.rsqrt(var + _EPS) * g_ref[...] + b_ref[...]
    d = _dot(xn, kk_ref[0], ((1,), (0,))) * scale + bias_ref[...]
    parts = []
    for off, w in segs:
        ds = d[:, off:off + w]
        m = jnp.max(ds, axis=1, keepdims=True)
        e = jnp.exp(ds - m)
        parts.append(e / jnp.sum(e, axis=1, keepdims=True))
    a = jnp.concatenate(parts, axis=1)
    out_ref[0] = _dot(a, vv_ref[0], ((1,), (0,)))


def _attention_fold(q_tokens, kk, vv, g, b, Ms, dh, bn):
    B, N, C = q_tokens.shape
    S = kk.shape[2]
    S1 = S // _HEADS
    nb = -(-N // bn)
    Np = nb * bn
    if Np != N:
        q_tokens = jnp.pad(q_tokens, ((0, 0), (0, Np - N), (0, 0)))
    scale = dh ** (-0.5)
    segs = []
    bias = jnp.zeros((S,), jnp.float32)
    for h in range(_HEADS):
        off = 0
        for M in Ms:
            Mp = _pad128(M)
            segs.append((h * S1 + off, Mp))
            if Mp != M:
                bias = bias.at[h * S1 + off + M:h * S1 + off + Mp].set(-1e30)
            off += Mp
    out = pl.pallas_call(
        functools.partial(_attn_fold_body, scale, segs),
        grid=(B, nb),
        in_specs=[
            pl.BlockSpec((1, bn, C), lambda bb, nn: (bb, nn, 0)),
            pl.BlockSpec((1, C), lambda bb, nn: (0, 0)),
            pl.BlockSpec((1, C), lambda bb, nn: (0, 0)),
            pl.BlockSpec((1, C, S), lambda bb, nn: (bb, 0, 0)),
            pl.BlockSpec((1, S, C), lambda bb, nn: (bb, 0, 0)),
            pl.BlockSpec((1, S), lambda bb, nn: (0, 0)),
        ],
        out_specs=pl.BlockSpec((1, bn, C), lambda bb, nn: (bb, nn, 0)),
        out_shape=jax.ShapeDtypeStruct((B, Np, C), jnp.float32),
    )(q_tokens, g.reshape(1, C), b.reshape(1, C), kk, vv,
      bias.reshape(1, S))
    return out[:, :N, :]


def _attn_body(dh, scale, q_ref, k0, v0, k1, v1, k2, v2,
               g_ref, b_ref, wq_ref, wot_ref, out_ref):
    ks = (k0, k1, k2)
    vs = (v0, v1, v2)
    x = _ln(q_ref[0], g_ref[0], b_ref[0])
    BN, C = x.shape
    acc = jnp.zeros((BN, C), jnp.float32)
    for h in range(_HEADS):
        wq_h = wq_ref[h * dh:(h + 1) * dh, :]
        qh = _dot(x, wq_h, ((1,), (1,)))
        for c in range(3):
            kh = ks[c][0][h * dh:(h + 1) * dh, :]
            d = _dot(qh, kh, ((1,), (0,))) * scale
            d = d - jnp.max(d, axis=1, keepdims=True)
            e = jnp.exp(d)
            a = e / jnp.sum(e, axis=1, keepdims=True)
            vh = vs[c][0][h * dh:(h + 1) * dh, :]
            o = _dot(a, vh, ((1,), (1,)))
            wsl = wot_ref[(h * 3 + c) * dh:(h * 3 + c + 1) * dh, :]
            acc = acc + _dot(o, wsl, ((1,), (0,)))
    out_ref[0] = acc


def _attention(q_tokens, kts, vts, g, b, wq, wout, dh, bn):
    B, N, C = q_tokens.shape
    inner = _HEADS * dh
    nb = -(-N // bn)
    Np = nb * bn
    if Np != N:
        q_tokens = jnp.pad(q_tokens, ((0, 0), (0, Np - N), (0, 0)))
    scale = dh ** (-0.5)
    kv_specs = []
    for kt in kts:
        M = kt.shape[2]
        kv_specs.append(pl.BlockSpec((1, inner, M), lambda bb, nn: (bb, 0, 0)))
        kv_specs.append(pl.BlockSpec((1, inner, M), lambda bb, nn: (bb, 0, 0)))
    out = pl.pallas_call(
        functools.partial(_attn_body, dh, scale),
        grid=(B, nb),
        in_specs=[pl.BlockSpec((1, bn, C), lambda bb, nn: (bb, nn, 0))]
        + kv_specs
        + [
            pl.BlockSpec((1, C), lambda bb, nn: (0, 0)),
            pl.BlockSpec((1, C), lambda bb, nn: (0, 0)),
            pl.BlockSpec((inner, C), lambda bb, nn: (0, 0)),
            pl.BlockSpec((3 * inner, C), lambda bb, nn: (0, 0)),
        ],
        out_specs=pl.BlockSpec((1, bn, C), lambda bb, nn: (bb, nn, 0)),
        out_shape=jax.ShapeDtypeStruct((B, Np, C), jnp.float32),
    )(q_tokens, kts[0], vts[0], kts[1], vts[1], kts[2], vts[2],
      g.reshape(1, C), b.reshape(1, C), wq, wout.T)
    return out[:, :N, :]


_BN = [512, 400, 576, 144]


def _smla(xs, blocks):
    B = xs[0].shape[0]
    tmp_q = []
    tmp_skips = []
    for i, x in enumerate(xs):
        C, H, W = _SHAPES[i]
        q = x.reshape(B, C, H * W).transpose(0, 2, 1)
        tmp_q.append(q)
        if _KS[i] == 1:
            tmp_skips.append(q)
        else:
            u = _unfold(x, _KS[i])
            tmp_skips.append(_select_tokens(u, _TOPK[i]))
    outs = []
    for idx in range(4):
        ap = blocks[idx]
        Ci = _CS[idx]
        inner = _HEADS * Ci
        skips = [tmp_skips[j] for j in range(4) if j != idx]
        if idx < 3:
            kk, vv = _fold_kv(skips, ap['kvn_g'], ap['kvn_b'], ap['Wkv'],
                              ap['Wq'], ap['Wout'], Ci)
            o = _attention_fold(tmp_q[idx], kk, vv, ap['qn_g'], ap['qn_b'],
                                [s.shape[1] for s in skips], Ci, _BN[idx])
        else:
            kts, vts = [], []
            for c in range(3):
                kt, vt = _kv_project(skips[c], ap['kvn_g'][c],
                                     ap['kvn_b'][c], ap['Wkv'][c], inner)
                kts.append(kt)
                vts.append(vt)
            o = _attention(tmp_q[idx], kts, vts, ap['qn_g'], ap['qn_b'],
                           ap['Wq'], ap['Wout'], Ci, _BN[idx])
        C, H, W = _SHAPES[idx]
        outs.append(o.transpose(0, 2, 1).reshape(B, C, H, W))
    return outs


def kernel(x0, x1, x2, x3, params):
    xs = [x0, x1, x2, x3]
    for t in range(_T_NUM):
        xs = _smla(xs, params[t])
    return tuple(xs)


# speedup vs baseline: 1.2018x; 1.0517x over previous
import functools

import jax
import jax.numpy as jnp
from jax.experimental import pallas as pl

_CS = [64, 128, 256, 512]
_SHAPES = [(64, 70, 70), (128, 40, 40), (256, 24, 24), (512, 12, 12)]
_KS = [7, 5, 3, 1]
_TOPK = [4, 3, 3, 1]
_HEADS = 4
_T_NUM = 2
_EPS = 1e-5
_PREC = jax.lax.Precision.DEFAULT


def _dot(a, b, dims):
    return jax.lax.dot_general(a, b, (dims, ((), ())), precision=_PREC,
                               preferred_element_type=jnp.float32)


def _ln(x, g, b):
    mu = jnp.mean(x, axis=-1, keepdims=True)
    var = jnp.mean((x - mu) ** 2, axis=-1, keepdims=True)
    return (x - mu) / jnp.sqrt(var + _EPS) * g + b


def _select_body(K, u_ref, out_ref):
    u = u_ref[0]
    L, v, C = u.shape
    ub = u.astype(jnp.bfloat16).astype(jnp.float32)
    s = jnp.sum(ub, axis=1)
    rel = jnp.sum(ub * s[:, None, :], axis=2) / v
    iota = jax.lax.broadcasted_iota(jnp.int32, (L, v), 1)
    picks = []
    for _ in range(K):
        mn = jnp.min(rel, axis=1, keepdims=True)
        cand = jnp.where(rel == mn, iota, v)
        first = jnp.min(cand, axis=1, keepdims=True)
        onehot = iota == first
        w = onehot.astype(jnp.float32)
        picks.append(jnp.sum(u * w[:, :, None], axis=1))
        rel = jnp.where(onehot, jnp.inf, rel)
    sel = jnp.concatenate([p[:, None, :] for p in picks], axis=1)
    out_ref[0] = sel


def _select_tokens_all(us, Ks):
    B = us[0].shape[0]

    def body(*refs):
        n = len(us)
        for i in range(n):
            _select_body(Ks[i], refs[i], refs[n + i])

    out = pl.pallas_call(
        body,
        grid=(B,),
        in_specs=[pl.BlockSpec((1,) + u.shape[1:], lambda b: (b, 0, 0, 0))
                  for u in us],
        out_specs=[pl.BlockSpec((1, u.shape[1], K, u.shape[3]),
                                lambda b: (b, 0, 0, 0))
                   for u, K in zip(us, Ks)],
        out_shape=[jax.ShapeDtypeStruct((B, u.shape[1], K, u.shape[3]),
                                        jnp.float32)
                   for u, K in zip(us, Ks)],
    )(*us)
    return [o.reshape(B, o.shape[1] * o.shape[2], o.shape[3]) for o in out]


def _unfold_tok(xt, H, W, k):
    B, N, C = xt.shape
    Hp, Wp = H // k, W // k
    u = xt.reshape(B, Hp, k, Wp, k, C)
    return jnp.transpose(u, (0, 1, 3, 2, 4, 5)).reshape(B, Hp * Wp, k * k, C)


def _kv_project_all(skips, gs, bs, ws, inner):
    B = skips[0].shape[0]

    def body(*refs):
        for c in range(3):
            x_ref, g_ref, b_ref, w_ref = refs[4 * c:4 * c + 4]
            k_ref, v_ref = refs[12 + 2 * c:12 + 2 * c + 2]
            x = _ln(x_ref[0], g_ref[0], b_ref[0])
            kvt = _dot(w_ref[...], x, ((1,), (1,)))
            k_ref[0] = kvt[:inner, :]
            v_ref[0] = kvt[inner:, :]

    in_specs = []
    args = []
    out_specs = []
    out_shape = []
    for c in range(3):
        M, Cj = skips[c].shape[1], skips[c].shape[2]
        in_specs += [
            pl.BlockSpec((1, M, Cj), lambda bb: (bb, 0, 0)),
            pl.BlockSpec((1, Cj), lambda bb: (0, 0)),
            pl.BlockSpec((1, Cj), lambda bb: (0, 0)),
            pl.BlockSpec((2 * inner, Cj), lambda bb: (0, 0)),
        ]
        args += [skips[c], gs[c].reshape(1, Cj), bs[c].reshape(1, Cj), ws[c]]
        out_specs += [pl.BlockSpec((1, inner, M), lambda bb: (bb, 0, 0))] * 2
        out_shape += [jax.ShapeDtypeStruct((B, inner, M), jnp.float32)] * 2
    res = pl.pallas_call(
        body, grid=(B,), in_specs=in_specs, out_specs=out_specs,
        out_shape=out_shape,
    )(*args)
    return [res[0], res[2], res[4]], [res[1], res[3], res[5]]


def _pad128(m):
    return -(-m // 128) * 128


def _fold_body(dh, Ms, x_refs, g_refs, b_refs, w_refs, wq_ref, wot_ref,
               kk_ref, vv_ref):
    inner = _HEADS * dh
    S1 = sum(_pad128(m) for m in Ms)
    off = 0
    for c in range(3):
        M = Ms[c]
        Mp = _pad128(M)
        x = _ln(x_refs[c][0], g_refs[c][0], b_refs[c][0])
        kvt = _dot(w_refs[c][...], x, ((1,), (1,)))
        for h in range(_HEADS):
            kh = kvt[h * dh:(h + 1) * dh, :]
            wqh = wq_ref[h * dh:(h + 1) * dh, :]
            kk = _dot(wqh, kh, ((0,), (0,)))
            vh = kvt[inner + h * dh:inner + (h + 1) * dh, :]
            wth = wot_ref[(h * 3 + c) * dh:(h * 3 + c + 1) * dh, :]
            vv = _dot(vh, wth, ((0,), (0,)))
            if Mp != M:
                C = kk.shape[0]
                kk = jnp.concatenate(
                    [kk, jnp.zeros((C, Mp - M), jnp.float32)], axis=1)
                vv = jnp.concatenate(
                    [vv, jnp.zeros((Mp - M, C), jnp.float32)], axis=0)
            kk_ref[0, :, h * S1 + off:h * S1 + off + Mp] = kk
            vv_ref[0, h * S1 + off:h * S1 + off + Mp, :] = vv
        off += Mp


def _fold_kv_multi(cfgs):
    B = cfgs[0][0][0].shape[0]
    in_specs = []
    args = []
    out_specs = []
    out_shape = []
    metas = []
    for (skips, gs, bs, ws, wq, wout, dh) in cfgs:
        C = wq.shape[1]
        inner = _HEADS * dh
        Ms = [s.shape[1] for s in skips]
        S = _HEADS * sum(_pad128(m) for m in Ms)
        metas.append((dh, Ms))
        for c in range(3):
            M, Cj = skips[c].shape[1], skips[c].shape[2]
            in_specs += [
                pl.BlockSpec((1, M, Cj), lambda bb: (bb, 0, 0)),
                pl.BlockSpec((1, Cj), lambda bb: (0, 0)),
                pl.BlockSpec((1, Cj), lambda bb: (0, 0)),
                pl.BlockSpec((2 * inner, Cj), lambda bb: (0, 0)),
            ]
            args += [skips[c], gs[c].reshape(1, Cj), bs[c].reshape(1, Cj),
                     ws[c]]
        in_specs += [
            pl.BlockSpec((inner, C), lambda bb: (0, 0)),
            pl.BlockSpec((3 * inner, C), lambda bb: (0, 0)),
        ]
        args += [wq, wout.T]
        out_specs += [
            pl.BlockSpec((1, C, S), lambda bb: (bb, 0, 0)),
            pl.BlockSpec((1, S, C), lambda bb: (bb, 0, 0)),
        ]
        out_shape += [
            jax.ShapeDtypeStruct((B, C, S), jnp.float32),
            jax.ShapeDtypeStruct((B, S, C), jnp.float32),
        ]

    n_in = len(in_specs)

    def body(*refs):
        pos = 0
        for i, (dh, Ms) in enumerate(metas):
            r = refs[pos:pos + 14]
            _fold_body(dh, Ms, (r[0], r[4], r[8]), (r[1], r[5], r[9]),
                       (r[2], r[6], r[10]), (r[3], r[7], r[11]),
                       r[12], r[13],
                       refs[n_in + 2 * i], refs[n_in + 2 * i + 1])
            pos += 14

    res = pl.pallas_call(
        body, grid=(B,), in_specs=in_specs, out_specs=out_specs,
        out_shape=out_shape,
    )(*args)
    return [(res[2 * i], res[2 * i + 1]) for i in range(len(cfgs))]


def _attn_fold_body(scale, segs, x_ref, g_ref, b_ref, kk_ref, vv_ref,
                    bias_ref, out_ref):
    x = x_ref[0]
    BN, C = x.shape
    one = jnp.ones((C, 1), jnp.float32)
    s1 = _dot(x, one, ((1,), (0,)))
    s2 = _dot(x * x, one, ((1,), (0,)))
    mu = s1 / C
    var = s2 / C - mu * mu
    xn = (x - mu) * jax.lax.rsqrt(var + _EPS) * g_ref[...] + b_ref[...]
    d = _dot(xn, kk_ref[0], ((1,), (0,))) * scale + bias_ref[...]
    parts = []
    for off, w in segs:
        ds = d[:, off:off + w]
        m = jnp.max(ds, axis=1, keepdims=True)
        e = jnp.exp(ds - m)
        parts.append(e / jnp.sum(e, axis=1, keepdims=True))
    a = jnp.concatenate(parts, axis=1)
    out_ref[0] = _dot(a, vv_ref[0], ((1,), (0,)))


def _attention_fold(q_tokens, kk, vv, g, b, Ms, dh, bn):
    B, N, C = q_tokens.shape
    S = kk.shape[2]
    S1 = S // _HEADS
    nb = -(-N // bn)
    Np = nb * bn
    if Np != N:
        q_tokens = jnp.pad(q_tokens, ((0, 0), (0, Np - N), (0, 0)))
    scale = dh ** (-0.5)
    segs = []
    bias = jnp.zeros((S,), jnp.float32)
    for h in range(_HEADS):
        off = 0
        for M in Ms:
            Mp = _pad128(M)
            segs.append((h * S1 + off, Mp))
            if Mp != M:
                bias = bias.at[h * S1 + off + M:h * S1 + off + Mp].set(-1e30)
            off += Mp
    out = pl.pallas_call(
        functools.partial(_attn_fold_body, scale, segs),
        grid=(B, nb),
        in_specs=[
            pl.BlockSpec((1, bn, C), lambda bb, nn: (bb, nn, 0)),
            pl.BlockSpec((1, C), lambda bb, nn: (0, 0)),
            pl.BlockSpec((1, C), lambda bb, nn: (0, 0)),
            pl.BlockSpec((1, C, S), lambda bb, nn: (bb, 0, 0)),
            pl.BlockSpec((1, S, C), lambda bb, nn: (bb, 0, 0)),
            pl.BlockSpec((1, S), lambda bb, nn: (0, 0)),
        ],
        out_specs=pl.BlockSpec((1, bn, C), lambda bb, nn: (bb, nn, 0)),
        out_shape=jax.ShapeDtypeStruct((B, Np, C), jnp.float32),
    )(q_tokens, g.reshape(1, C), b.reshape(1, C), kk, vv,
      bias.reshape(1, S))
    return out[:, :N, :]


def _attn_body(dh, scale, q_ref, k0, v0, k1, v1, k2, v2,
               g_ref, b_ref, wq_ref, wot_ref, out_ref):
    ks = (k0, k1, k2)
    vs = (v0, v1, v2)
    x = _ln(q_ref[0], g_ref[0], b_ref[0])
    BN, C = x.shape
    acc = jnp.zeros((BN, C), jnp.float32)
    for h in range(_HEADS):
        wq_h = wq_ref[h * dh:(h + 1) * dh, :]
        qh = _dot(x, wq_h, ((1,), (1,)))
        for c in range(3):
            kh = ks[c][0][h * dh:(h + 1) * dh, :]
            d = _dot(qh, kh, ((1,), (0,))) * scale
            d = d - jnp.max(d, axis=1, keepdims=True)
            e = jnp.exp(d)
            a = e / jnp.sum(e, axis=1, keepdims=True)
            vh = vs[c][0][h * dh:(h + 1) * dh, :]
            o = _dot(a, vh, ((1,), (1,)))
            wsl = wot_ref[(h * 3 + c) * dh:(h * 3 + c + 1) * dh, :]
            acc = acc + _dot(o, wsl, ((1,), (0,)))
    out_ref[0] = acc


def _attention(q_tokens, kts, vts, g, b, wq, wout, dh, bn):
    B, N, C = q_tokens.shape
    inner = _HEADS * dh
    nb = -(-N // bn)
    Np = nb * bn
    if Np != N:
        q_tokens = jnp.pad(q_tokens, ((0, 0), (0, Np - N), (0, 0)))
    scale = dh ** (-0.5)
    kv_specs = []
    for kt in kts:
        M = kt.shape[2]
        kv_specs.append(pl.BlockSpec((1, inner, M), lambda bb, nn: (bb, 0, 0)))
        kv_specs.append(pl.BlockSpec((1, inner, M), lambda bb, nn: (bb, 0, 0)))
    out = pl.pallas_call(
        functools.partial(_attn_body, dh, scale),
        grid=(B, nb),
        in_specs=[pl.BlockSpec((1, bn, C), lambda bb, nn: (bb, nn, 0))]
        + kv_specs
        + [
            pl.BlockSpec((1, C), lambda bb, nn: (0, 0)),
            pl.BlockSpec((1, C), lambda bb, nn: (0, 0)),
            pl.BlockSpec((inner, C), lambda bb, nn: (0, 0)),
            pl.BlockSpec((3 * inner, C), lambda bb, nn: (0, 0)),
        ],
        out_specs=pl.BlockSpec((1, bn, C), lambda bb, nn: (bb, nn, 0)),
        out_shape=jax.ShapeDtypeStruct((B, Np, C), jnp.float32),
    )(q_tokens, kts[0], vts[0], kts[1], vts[1], kts[2], vts[2],
      g.reshape(1, C), b.reshape(1, C), wq, wout.T)
    return out[:, :N, :]


_BN = [512, 400, 576, 144]


def _smla(xs, blocks):
    B = xs[0].shape[0]
    us = [_unfold_tok(xs[i], _SHAPES[i][1], _SHAPES[i][2], _KS[i])
          for i in range(3)]
    tmp_skips = _select_tokens_all(us, _TOPK[:3]) + [xs[3]]
    folded = _fold_kv_multi([
        ([tmp_skips[j] for j in range(4) if j != idx],
         blocks[idx]['kvn_g'], blocks[idx]['kvn_b'], blocks[idx]['Wkv'],
         blocks[idx]['Wq'], blocks[idx]['Wout'], _CS[idx])
        for idx in range(3)])
    kts3, vts3 = _kv_project_all(tmp_skips[:3], blocks[3]['kvn_g'],
                                 blocks[3]['kvn_b'], blocks[3]['Wkv'],
                                 _HEADS * _CS[3])
    outs = []
    for idx in range(3):
        ap = blocks[idx]
        kk, vv = folded[idx]
        Ms = [tmp_skips[j].shape[1] for j in range(4) if j != idx]
        outs.append(_attention_fold(xs[idx], kk, vv, ap['qn_g'], ap['qn_b'],
                                    Ms, _CS[idx], _BN[idx]))
    ap = blocks[3]
    outs.append(_attention(xs[3], kts3, vts3, ap['qn_g'], ap['qn_b'],
                           ap['Wq'], ap['Wout'], _CS[3], _BN[3]))
    return outs


def kernel(x0, x1, x2, x3, params):
    B = x0.shape[0]
    xs = []
    for i, x in enumerate([x0, x1, x2, x3]):
        C, H, W = _SHAPES[i]
        xs.append(x.reshape(B, C, H * W).transpose(0, 2, 1))
    for t in range(_T_NUM):
        xs = _smla(xs, params[t])
    outs = []
    for i, o in enumerate(xs):
        C, H, W = _SHAPES[i]
        outs.append(o.transpose(0, 2, 1).reshape(B, C, H, W))
    return tuple(outs)


# speedup vs baseline: 1.2483x; 1.0387x over previous
import functools

import jax
import jax.numpy as jnp
from jax.experimental import pallas as pl

_CS = [64, 128, 256, 512]
_SHAPES = [(64, 70, 70), (128, 40, 40), (256, 24, 24), (512, 12, 12)]
_KS = [7, 5, 3, 1]
_TOPK = [4, 3, 3, 1]
_HEADS = 4
_T_NUM = 2
_EPS = 1e-5
_PREC = jax.lax.Precision.DEFAULT


def _dot(a, b, dims):
    return jax.lax.dot_general(a, b, (dims, ((), ())), precision=_PREC,
                               preferred_element_type=jnp.float32)


def _ln(x, g, b):
    mu = jnp.mean(x, axis=-1, keepdims=True)
    var = jnp.mean((x - mu) ** 2, axis=-1, keepdims=True)
    return (x - mu) / jnp.sqrt(var + _EPS) * g + b


def _select_body(K, u_ref, out_ref):
    u = u_ref[0]
    L, v, C = u.shape
    ub = u.astype(jnp.bfloat16).astype(jnp.float32)
    s = jnp.sum(ub, axis=1)
    rel = jnp.sum(ub * s[:, None, :], axis=2) / v
    iota = jax.lax.broadcasted_iota(jnp.int32, (L, v), 1)
    picks = []
    for _ in range(K):
        mn = jnp.min(rel, axis=1, keepdims=True)
        cand = jnp.where(rel == mn, iota, v)
        first = jnp.min(cand, axis=1, keepdims=True)
        onehot = iota == first
        w = onehot.astype(jnp.float32)
        picks.append(jnp.sum(u * w[:, :, None], axis=1))
        rel = jnp.where(onehot, jnp.inf, rel)
    sel = jnp.concatenate([p[:, None, :] for p in picks], axis=1)
    out_ref[0] = sel


def _select_tokens_all(us, Ks):
    B = us[0].shape[0]

    def body(*refs):
        n = len(us)
        for i in range(n):
            _select_body(Ks[i], refs[i], refs[n + i])

    out = pl.pallas_call(
        body,
        grid=(B,),
        in_specs=[pl.BlockSpec((1,) + u.shape[1:], lambda b: (b, 0, 0, 0))
                  for u in us],
        out_specs=[pl.BlockSpec((1, u.shape[1], K, u.shape[3]),
                                lambda b: (b, 0, 0, 0))
                   for u, K in zip(us, Ks)],
        out_shape=[jax.ShapeDtypeStruct((B, u.shape[1], K, u.shape[3]),
                                        jnp.float32)
                   for u, K in zip(us, Ks)],
    )(*us)
    return [o.reshape(B, o.shape[1] * o.shape[2], o.shape[3]) for o in out]


def _unfold_tok(xt, H, W, k):
    B, N, C = xt.shape
    Hp, Wp = H // k, W // k
    u = xt.reshape(B, Hp, k, Wp, k, C)
    return jnp.transpose(u, (0, 1, 3, 2, 4, 5)).reshape(B, Hp * Wp, k * k, C)


def _kv_project_all(skips, gs, bs, ws, inner):
    B = skips[0].shape[0]

    def body(*refs):
        for c in range(3):
            x_ref, g_ref, b_ref, w_ref = refs[4 * c:4 * c + 4]
            k_ref, v_ref = refs[12 + 2 * c:12 + 2 * c + 2]
            x = _ln(x_ref[0], g_ref[0], b_ref[0])
            kvt = _dot(w_ref[...], x, ((1,), (1,)))
            k_ref[0] = kvt[:inner, :]
            v_ref[0] = kvt[inner:, :]

    in_specs = []
    args = []
    out_specs = []
    out_shape = []
    for c in range(3):
        M, Cj = skips[c].shape[1], skips[c].shape[2]
        in_specs += [
            pl.BlockSpec((1, M, Cj), lambda bb: (bb, 0, 0)),
            pl.BlockSpec((1, Cj), lambda bb: (0, 0)),
            pl.BlockSpec((1, Cj), lambda bb: (0, 0)),
            pl.BlockSpec((2 * inner, Cj), lambda bb: (0, 0)),
        ]
        args += [skips[c], gs[c].reshape(1, Cj), bs[c].reshape(1, Cj), ws[c]]
        out_specs += [pl.BlockSpec((1, inner, M), lambda bb: (bb, 0, 0))] * 2
        out_shape += [jax.ShapeDtypeStruct((B, inner, M), jnp.float32)] * 2
    res = pl.pallas_call(
        body, grid=(B,), in_specs=in_specs, out_specs=out_specs,
        out_shape=out_shape,
    )(*args)
    return [res[0], res[2], res[4]], [res[1], res[3], res[5]]


def _pad128(m):
    return -(-m // 128) * 128


def _fold_body(dh, Ms, x_refs, g_refs, b_refs, w_refs, wq_ref, wot_ref,
               kk_ref, vv_ref):
    inner = _HEADS * dh
    S1 = sum(_pad128(m) for m in Ms)
    off = 0
    for c in range(3):
        M = Ms[c]
        Mp = _pad128(M)
        x = _ln(x_refs[c][0], g_refs[c][0], b_refs[c][0])
        kvt = _dot(w_refs[c][...], x, ((1,), (1,)))
        for h in range(_HEADS):
            kh = kvt[h * dh:(h + 1) * dh, :]
            wqh = wq_ref[h * dh:(h + 1) * dh, :]
            kk = _dot(wqh, kh, ((0,), (0,)))
            vh = kvt[inner + h * dh:inner + (h + 1) * dh, :]
            wth = wot_ref[(h * 3 + c) * dh:(h * 3 + c + 1) * dh, :]
            vv = _dot(vh, wth, ((0,), (0,)))
            if Mp != M:
                C = kk.shape[0]
                kk = jnp.concatenate(
                    [kk, jnp.zeros((C, Mp - M), jnp.float32)], axis=1)
                vv = jnp.concatenate(
                    [vv, jnp.zeros((Mp - M, C), jnp.float32)], axis=0)
            kk_ref[0, :, h * S1 + off:h * S1 + off + Mp] = kk
            vv_ref[0, h * S1 + off:h * S1 + off + Mp, :] = vv
        off += Mp


def _fold_kv_multi(cfgs):
    B = cfgs[0][0][0].shape[0]
    in_specs = []
    args = []
    out_specs = []
    out_shape = []
    metas = []
    for (skips, gs, bs, ws, wq, wout, dh) in cfgs:
        C = wq.shape[1]
        inner = _HEADS * dh
        Ms = [s.shape[1] for s in skips]
        S = _HEADS * sum(_pad128(m) for m in Ms)
        metas.append((dh, Ms))
        for c in range(3):
            M, Cj = skips[c].shape[1], skips[c].shape[2]
            in_specs += [
                pl.BlockSpec((1, M, Cj), lambda bb: (bb, 0, 0)),
                pl.BlockSpec((1, Cj), lambda bb: (0, 0)),
                pl.BlockSpec((1, Cj), lambda bb: (0, 0)),
                pl.BlockSpec((2 * inner, Cj), lambda bb: (0, 0)),
            ]
            args += [skips[c], gs[c].reshape(1, Cj), bs[c].reshape(1, Cj),
                     ws[c]]
        in_specs += [
            pl.BlockSpec((inner, C), lambda bb: (0, 0)),
            pl.BlockSpec((3 * inner, C), lambda bb: (0, 0)),
        ]
        args += [wq, wout.T]
        out_specs += [
            pl.BlockSpec((1, C, S), lambda bb: (bb, 0, 0)),
            pl.BlockSpec((1, S, C), lambda bb: (bb, 0, 0)),
        ]
        out_shape += [
            jax.ShapeDtypeStruct((B, C, S), jnp.float32),
            jax.ShapeDtypeStruct((B, S, C), jnp.float32),
        ]

    n_in = len(in_specs)

    def body(*refs):
        pos = 0
        for i, (dh, Ms) in enumerate(metas):
            r = refs[pos:pos + 14]
            _fold_body(dh, Ms, (r[0], r[4], r[8]), (r[1], r[5], r[9]),
                       (r[2], r[6], r[10]), (r[3], r[7], r[11]),
                       r[12], r[13],
                       refs[n_in + 2 * i], refs[n_in + 2 * i + 1])
            pos += 14

    res = pl.pallas_call(
        body, grid=(B,), in_specs=in_specs, out_specs=out_specs,
        out_shape=out_shape,
    )(*args)
    return [(res[2 * i], res[2 * i + 1]) for i in range(len(cfgs))]


def _attn_fold_body(scale, segs, x_ref, g_ref, b_ref, kk_ref, vv_ref,
                    bias_ref, out_ref):
    x = x_ref[0]
    BN, C = x.shape
    one = jnp.ones((C, 1), jnp.float32)
    s1 = _dot(x, one, ((1,), (0,)))
    s2 = _dot(x * x, one, ((1,), (0,)))
    mu = s1 / C
    var = s2 / C - mu * mu
    xn = (x - mu) * jax.lax.rsqrt(var + _EPS) * g_ref[...] + b_ref[...]
    d = _dot(xn, kk_ref[0], ((1,), (0,))) * scale + bias_ref[...]
    e = jnp.exp(d)
    parts = []
    for off, w in segs:
        es = e[:, off:off + w]
        parts.append(es / jnp.sum(es, axis=1, keepdims=True))
    a = jnp.concatenate(parts, axis=1)
    out_ref[0] = _dot(a, vv_ref[0], ((1,), (0,)))


def _attention_fold(q_tokens, kk, vv, g, b, Ms, dh, bn):
    B, N, C = q_tokens.shape
    S = kk.shape[2]
    S1 = S // _HEADS
    nb = -(-N // bn)
    Np = nb * bn
    if Np != N:
        q_tokens = jnp.pad(q_tokens, ((0, 0), (0, Np - N), (0, 0)))
    scale = dh ** (-0.5)
    segs = []
    bias = jnp.zeros((S,), jnp.float32)
    for h in range(_HEADS):
        off = 0
        for M in Ms:
            Mp = _pad128(M)
            segs.append((h * S1 + off, Mp))
            if Mp != M:
                bias = bias.at[h * S1 + off + M:h * S1 + off + Mp].set(-1e30)
            off += Mp
    out = pl.pallas_call(
        functools.partial(_attn_fold_body, scale, segs),
        grid=(B, nb),
        in_specs=[
            pl.BlockSpec((1, bn, C), lambda bb, nn: (bb, nn, 0)),
            pl.BlockSpec((1, C), lambda bb, nn: (0, 0)),
            pl.BlockSpec((1, C), lambda bb, nn: (0, 0)),
            pl.BlockSpec((1, C, S), lambda bb, nn: (bb, 0, 0)),
            pl.BlockSpec((1, S, C), lambda bb, nn: (bb, 0, 0)),
            pl.BlockSpec((1, S), lambda bb, nn: (0, 0)),
        ],
        out_specs=pl.BlockSpec((1, bn, C), lambda bb, nn: (bb, nn, 0)),
        out_shape=jax.ShapeDtypeStruct((B, Np, C), jnp.float32),
    )(q_tokens, g.reshape(1, C), b.reshape(1, C), kk, vv,
      bias.reshape(1, S))
    return out[:, :N, :]


def _attn_body(dh, scale, q_ref, k0, v0, k1, v1, k2, v2,
               g_ref, b_ref, wq_ref, wot_ref, out_ref):
    ks = (k0, k1, k2)
    vs = (v0, v1, v2)
    x = _ln(q_ref[0], g_ref[0], b_ref[0])
    BN, C = x.shape
    acc = jnp.zeros((BN, C), jnp.float32)
    for h in range(_HEADS):
        wq_h = wq_ref[h * dh:(h + 1) * dh, :]
        qh = _dot(x, wq_h, ((1,), (1,)))
        for c in range(3):
            kh = ks[c][0][h * dh:(h + 1) * dh, :]
            d = _dot(qh, kh, ((1,), (0,))) * scale
            d = d - jnp.max(d, axis=1, keepdims=True)
            e = jnp.exp(d)
            a = e / jnp.sum(e, axis=1, keepdims=True)
            vh = vs[c][0][h * dh:(h + 1) * dh, :]
            o = _dot(a, vh, ((1,), (1,)))
            wsl = wot_ref[(h * 3 + c) * dh:(h * 3 + c + 1) * dh, :]
            acc = acc + _dot(o, wsl, ((1,), (0,)))
    out_ref[0] = acc


def _attention(q_tokens, kts, vts, g, b, wq, wout, dh, bn):
    B, N, C = q_tokens.shape
    inner = _HEADS * dh
    nb = -(-N // bn)
    Np = nb * bn
    if Np != N:
        q_tokens = jnp.pad(q_tokens, ((0, 0), (0, Np - N), (0, 0)))
    scale = dh ** (-0.5)
    kv_specs = []
    for kt in kts:
        M = kt.shape[2]
        kv_specs.append(pl.BlockSpec((1, inner, M), lambda bb, nn: (bb, 0, 0)))
        kv_specs.append(pl.BlockSpec((1, inner, M), lambda bb, nn: (bb, 0, 0)))
    out = pl.pallas_call(
        functools.partial(_attn_body, dh, scale),
        grid=(B, nb),
        in_specs=[pl.BlockSpec((1, bn, C), lambda bb, nn: (bb, nn, 0))]
        + kv_specs
        + [
            pl.BlockSpec((1, C), lambda bb, nn: (0, 0)),
            pl.BlockSpec((1, C), lambda bb, nn: (0, 0)),
            pl.BlockSpec((inner, C), lambda bb, nn: (0, 0)),
            pl.BlockSpec((3 * inner, C), lambda bb, nn: (0, 0)),
        ],
        out_specs=pl.BlockSpec((1, bn, C), lambda bb, nn: (bb, nn, 0)),
        out_shape=jax.ShapeDtypeStruct((B, Np, C), jnp.float32),
    )(q_tokens, kts[0], vts[0], kts[1], vts[1], kts[2], vts[2],
      g.reshape(1, C), b.reshape(1, C), wq, wout.T)
    return out[:, :N, :]


_BN = [1024, 800, 576, 144]


def _smla(xs, blocks):
    B = xs[0].shape[0]
    us = [_unfold_tok(xs[i], _SHAPES[i][1], _SHAPES[i][2], _KS[i])
          for i in range(3)]
    tmp_skips = _select_tokens_all(us, _TOPK[:3]) + [xs[3]]
    folded = _fold_kv_multi([
        ([tmp_skips[j] for j in range(4) if j != idx],
         blocks[idx]['kvn_g'], blocks[idx]['kvn_b'], blocks[idx]['Wkv'],
         blocks[idx]['Wq'], blocks[idx]['Wout'], _CS[idx])
        for idx in range(3)])
    kts3, vts3 = _kv_project_all(tmp_skips[:3], blocks[3]['kvn_g'],
                                 blocks[3]['kvn_b'], blocks[3]['Wkv'],
                                 _HEADS * _CS[3])
    outs = []
    for idx in range(3):
        ap = blocks[idx]
        kk, vv = folded[idx]
        Ms = [tmp_skips[j].shape[1] for j in range(4) if j != idx]
        outs.append(_attention_fold(xs[idx], kk, vv, ap['qn_g'], ap['qn_b'],
                                    Ms, _CS[idx], _BN[idx]))
    ap = blocks[3]
    outs.append(_attention(xs[3], kts3, vts3, ap['qn_g'], ap['qn_b'],
                           ap['Wq'], ap['Wout'], _CS[3], _BN[3]))
    return outs


def kernel(x0, x1, x2, x3, params):
    B = x0.shape[0]
    xs = []
    for i, x in enumerate([x0, x1, x2, x3]):
        C, H, W = _SHAPES[i]
        xs.append(x.reshape(B, C, H * W).transpose(0, 2, 1))
    for t in range(_T_NUM):
        xs = _smla(xs, params[t])
    outs = []
    for i, o in enumerate(xs):
        C, H, W = _SHAPES[i]
        outs.append(o.transpose(0, 2, 1).reshape(B, C, H, W))
    return tuple(outs)


# speedup vs baseline: 1.3280x; 1.0639x over previous
import functools

import jax
import jax.numpy as jnp
from jax.experimental import pallas as pl

_CS = [64, 128, 256, 512]
_SHAPES = [(64, 70, 70), (128, 40, 40), (256, 24, 24), (512, 12, 12)]
_KS = [7, 5, 3, 1]
_TOPK = [4, 3, 3, 1]
_HEADS = 4
_T_NUM = 2
_EPS = 1e-5
_PREC = jax.lax.Precision.DEFAULT


def _dot(a, b, dims):
    return jax.lax.dot_general(a, b, (dims, ((), ())), precision=_PREC,
                               preferred_element_type=jnp.float32)


def _ln(x, g, b):
    mu = jnp.mean(x, axis=-1, keepdims=True)
    var = jnp.mean((x - mu) ** 2, axis=-1, keepdims=True)
    return (x - mu) / jnp.sqrt(var + _EPS) * g + b


def _select_body(K, u_ref, out_ref):
    u = u_ref[0]
    L, v, C = u.shape
    ub = u.astype(jnp.bfloat16).astype(jnp.float32)
    s = jnp.sum(ub, axis=1)
    rel = jnp.sum(ub * s[:, None, :], axis=2) / v
    iota = jax.lax.broadcasted_iota(jnp.int32, (L, v), 1)
    picks = []
    for _ in range(K):
        mn = jnp.min(rel, axis=1, keepdims=True)
        cand = jnp.where(rel == mn, iota, v)
        first = jnp.min(cand, axis=1, keepdims=True)
        onehot = iota == first
        w = onehot.astype(jnp.float32)
        picks.append(jnp.sum(u * w[:, :, None], axis=1))
        rel = jnp.where(onehot, jnp.inf, rel)
    sel = jnp.concatenate([p[:, None, :] for p in picks], axis=1)
    out_ref[0] = sel


def _select_tokens_all(us, Ks):
    B = us[0].shape[0]

    def body(*refs):
        n = len(us)
        for i in range(n):
            _select_body(Ks[i], refs[i], refs[n + i])

    out = pl.pallas_call(
        body,
        grid=(B,),
        in_specs=[pl.BlockSpec((1,) + u.shape[1:], lambda b: (b, 0, 0, 0))
                  for u in us],
        out_specs=[pl.BlockSpec((1, u.shape[1], K, u.shape[3]),
                                lambda b: (b, 0, 0, 0))
                   for u, K in zip(us, Ks)],
        out_shape=[jax.ShapeDtypeStruct((B, u.shape[1], K, u.shape[3]),
                                        jnp.float32)
                   for u, K in zip(us, Ks)],
    )(*us)
    return [o.reshape(B, o.shape[1] * o.shape[2], o.shape[3]) for o in out]


def _unfold_tok(xt, H, W, k):
    B, N, C = xt.shape
    Hp, Wp = H // k, W // k
    u = xt.reshape(B, Hp, k, Wp, k, C)
    return jnp.transpose(u, (0, 1, 3, 2, 4, 5)).reshape(B, Hp * Wp, k * k, C)


def _kv_project_all(skips, gs, bs, ws, inner):
    B = skips[0].shape[0]

    def body(*refs):
        for c in range(3):
            x_ref, g_ref, b_ref, w_ref = refs[4 * c:4 * c + 4]
            k_ref, v_ref = refs[12 + 2 * c:12 + 2 * c + 2]
            x = _ln(x_ref[0], g_ref[0], b_ref[0])
            kvt = _dot(w_ref[...], x, ((1,), (1,)))
            k_ref[0] = kvt[:inner, :].astype(jnp.bfloat16)
            v_ref[0] = kvt[inner:, :].astype(jnp.bfloat16)

    in_specs = []
    args = []
    out_specs = []
    out_shape = []
    for c in range(3):
        M, Cj = skips[c].shape[1], skips[c].shape[2]
        in_specs += [
            pl.BlockSpec((1, M, Cj), lambda bb: (bb, 0, 0)),
            pl.BlockSpec((1, Cj), lambda bb: (0, 0)),
            pl.BlockSpec((1, Cj), lambda bb: (0, 0)),
            pl.BlockSpec((2 * inner, Cj), lambda bb: (0, 0)),
        ]
        args += [skips[c], gs[c].reshape(1, Cj), bs[c].reshape(1, Cj), ws[c]]
        out_specs += [pl.BlockSpec((1, inner, M), lambda bb: (bb, 0, 0))] * 2
        out_shape += [jax.ShapeDtypeStruct((B, inner, M), jnp.bfloat16)] * 2
    res = pl.pallas_call(
        body, grid=(B,), in_specs=in_specs, out_specs=out_specs,
        out_shape=out_shape,
    )(*args)
    return [res[0], res[2], res[4]], [res[1], res[3], res[5]]


def _pad128(m):
    return -(-m // 128) * 128


def _fold_body(dh, Ms, x_refs, g_refs, b_refs, w_refs, wq_ref, wot_ref,
               kk_ref, vv_ref):
    inner = _HEADS * dh
    S1 = sum(_pad128(m) for m in Ms)
    off = 0
    for c in range(3):
        M = Ms[c]
        Mp = _pad128(M)
        x = _ln(x_refs[c][0], g_refs[c][0], b_refs[c][0])
        kvt = _dot(w_refs[c][...], x, ((1,), (1,)))
        for h in range(_HEADS):
            kh = kvt[h * dh:(h + 1) * dh, :]
            wqh = wq_ref[h * dh:(h + 1) * dh, :]
            kk = _dot(wqh, kh, ((0,), (0,))).astype(jnp.bfloat16)
            vh = kvt[inner + h * dh:inner + (h + 1) * dh, :]
            wth = wot_ref[(h * 3 + c) * dh:(h * 3 + c + 1) * dh, :]
            vv = _dot(vh, wth, ((0,), (0,))).astype(jnp.bfloat16)
            if Mp != M:
                C = kk.shape[0]
                kk = jnp.concatenate(
                    [kk, jnp.zeros((C, Mp - M), jnp.bfloat16)], axis=1)
                vv = jnp.concatenate(
                    [vv, jnp.zeros((Mp - M, C), jnp.bfloat16)], axis=0)
            kk_ref[0, :, h * S1 + off:h * S1 + off + Mp] = kk
            vv_ref[0, h * S1 + off:h * S1 + off + Mp, :] = vv
        off += Mp


def _fold_kv_multi(cfgs):
    B = cfgs[0][0][0].shape[0]
    in_specs = []
    args = []
    out_specs = []
    out_shape = []
    metas = []
    for (skips, gs, bs, ws, wq, wout, dh) in cfgs:
        C = wq.shape[1]
        inner = _HEADS * dh
        Ms = [s.shape[1] for s in skips]
        S = _HEADS * sum(_pad128(m) for m in Ms)
        metas.append((dh, Ms))
        for c in range(3):
            M, Cj = skips[c].shape[1], skips[c].shape[2]
            in_specs += [
                pl.BlockSpec((1, M, Cj), lambda bb: (bb, 0, 0)),
                pl.BlockSpec((1, Cj), lambda bb: (0, 0)),
                pl.BlockSpec((1, Cj), lambda bb: (0, 0)),
                pl.BlockSpec((2 * inner, Cj), lambda bb: (0, 0)),
            ]
            args += [skips[c], gs[c].reshape(1, Cj), bs[c].reshape(1, Cj),
                     ws[c]]
        in_specs += [
            pl.BlockSpec((inner, C), lambda bb: (0, 0)),
            pl.BlockSpec((3 * inner, C), lambda bb: (0, 0)),
        ]
        args += [wq, wout.T]
        out_specs += [
            pl.BlockSpec((1, C, S), lambda bb: (bb, 0, 0)),
            pl.BlockSpec((1, S, C), lambda bb: (bb, 0, 0)),
        ]
        out_shape += [
            jax.ShapeDtypeStruct((B, C, S), jnp.bfloat16),
            jax.ShapeDtypeStruct((B, S, C), jnp.bfloat16),
        ]

    n_in = len(in_specs)

    def body(*refs):
        pos = 0
        for i, (dh, Ms) in enumerate(metas):
            r = refs[pos:pos + 14]
            _fold_body(dh, Ms, (r[0], r[4], r[8]), (r[1], r[5], r[9]),
                       (r[2], r[6], r[10]), (r[3], r[7], r[11]),
                       r[12], r[13],
                       refs[n_in + 2 * i], refs[n_in + 2 * i + 1])
            pos += 14

    res = pl.pallas_call(
        body, grid=(B,), in_specs=in_specs, out_specs=out_specs,
        out_shape=out_shape,
    )(*args)
    return [(res[2 * i], res[2 * i + 1]) for i in range(len(cfgs))]


def _attn_fold_body(scale, segs, x_ref, g_ref, b_ref, kk_ref, vv_ref,
                    bias_ref, out_ref):
    x = x_ref[0]
    BN, C = x.shape
    one = jnp.ones((C, 1), jnp.float32)
    s1 = _dot(x, one, ((1,), (0,)))
    s2 = _dot(x * x, one, ((1,), (0,)))
    mu = s1 / C
    var = s2 / C - mu * mu
    xn = (x - mu) * jax.lax.rsqrt(var + _EPS) * g_ref[...] + b_ref[...]
    d = _dot(xn.astype(jnp.bfloat16), kk_ref[0], ((1,), (0,)))
    d = d * scale + bias_ref[...]
    e = jnp.exp(d)
    parts = []
    for off, w in segs:
        es = e[:, off:off + w]
        parts.append(es / jnp.sum(es, axis=1, keepdims=True))
    a = jnp.concatenate(parts, axis=1)
    out_ref[0] = _dot(a.astype(jnp.bfloat16), vv_ref[0], ((1,), (0,)))


def _attention_fold(q_tokens, kk, vv, g, b, Ms, dh, bn):
    B, N, C = q_tokens.shape
    S = kk.shape[2]
    S1 = S // _HEADS
    nb = -(-N // bn)
    Np = nb * bn
    if Np != N:
        q_tokens = jnp.pad(q_tokens, ((0, 0), (0, Np - N), (0, 0)))
    scale = dh ** (-0.5)
    segs = []
    bias = jnp.zeros((S,), jnp.float32)
    for h in range(_HEADS):
        off = 0
        for M in Ms:
            Mp = _pad128(M)
            segs.append((h * S1 + off, Mp))
            if Mp != M:
                bias = bias.at[h * S1 + off + M:h * S1 + off + Mp].set(-1e30)
            off += Mp
    out = pl.pallas_call(
        functools.partial(_attn_fold_body, scale, segs),
        grid=(B, nb),
        in_specs=[
            pl.BlockSpec((1, bn, C), lambda bb, nn: (bb, nn, 0)),
            pl.BlockSpec((1, C), lambda bb, nn: (0, 0)),
            pl.BlockSpec((1, C), lambda bb, nn: (0, 0)),
            pl.BlockSpec((1, C, S), lambda bb, nn: (bb, 0, 0)),
            pl.BlockSpec((1, S, C), lambda bb, nn: (bb, 0, 0)),
            pl.BlockSpec((1, S), lambda bb, nn: (0, 0)),
        ],
        out_specs=pl.BlockSpec((1, bn, C), lambda bb, nn: (bb, nn, 0)),
        out_shape=jax.ShapeDtypeStruct((B, Np, C), jnp.float32),
    )(q_tokens, g.reshape(1, C), b.reshape(1, C), kk, vv,
      bias.reshape(1, S))
    return out[:, :N, :]


def _attn_body(dh, scale, q_ref, k0, v0, k1, v1, k2, v2,
               g_ref, b_ref, wq_ref, wot_ref, out_ref):
    ks = (k0, k1, k2)
    vs = (v0, v1, v2)
    x = _ln(q_ref[0], g_ref[0], b_ref[0])
    BN, C = x.shape
    acc = jnp.zeros((BN, C), jnp.float32)
    for h in range(_HEADS):
        wq_h = wq_ref[h * dh:(h + 1) * dh, :]
        qh = _dot(x, wq_h, ((1,), (1,)))
        for c in range(3):
            kh = ks[c][0][h * dh:(h + 1) * dh, :]
            d = _dot(qh.astype(jnp.bfloat16), kh, ((1,), (0,))) * scale
            d = d - jnp.max(d, axis=1, keepdims=True)
            e = jnp.exp(d)
            a = e / jnp.sum(e, axis=1, keepdims=True)
            vh = vs[c][0][h * dh:(h + 1) * dh, :]
            o = _dot(a.astype(jnp.bfloat16), vh, ((1,), (1,)))
            wsl = wot_ref[(h * 3 + c) * dh:(h * 3 + c + 1) * dh, :]
            acc = acc + _dot(o, wsl, ((1,), (0,)))
    out_ref[0] = acc


def _attention(q_tokens, kts, vts, g, b, wq, wout, dh, bn):
    B, N, C = q_tokens.shape
    inner = _HEADS * dh
    nb = -(-N // bn)
    Np = nb * bn
    if Np != N:
        q_tokens = jnp.pad(q_tokens, ((0, 0), (0, Np - N), (0, 0)))
    scale = dh ** (-0.5)
    kv_specs = []
    for kt in kts:
        M = kt.shape[2]
        kv_specs.append(pl.BlockSpec((1, inner, M), lambda bb, nn: (bb, 0, 0)))
        kv_specs.append(pl.BlockSpec((1, inner, M), lambda bb, nn: (bb, 0, 0)))
    out = pl.pallas_call(
        functools.partial(_attn_body, dh, scale),
        grid=(B, nb),
        in_specs=[pl.BlockSpec((1, bn, C), lambda bb, nn: (bb, nn, 0))]
        + kv_specs
        + [
            pl.BlockSpec((1, C), lambda bb, nn: (0, 0)),
            pl.BlockSpec((1, C), lambda bb, nn: (0, 0)),
            pl.BlockSpec((inner, C), lambda bb, nn: (0, 0)),
            pl.BlockSpec((3 * inner, C), lambda bb, nn: (0, 0)),
        ],
        out_specs=pl.BlockSpec((1, bn, C), lambda bb, nn: (bb, nn, 0)),
        out_shape=jax.ShapeDtypeStruct((B, Np, C), jnp.float32),
    )(q_tokens, kts[0], vts[0], kts[1], vts[1], kts[2], vts[2],
      g.reshape(1, C), b.reshape(1, C), wq, wout.T)
    return out[:, :N, :]


_BN = [1024, 800, 576, 144]


def _smla(xs, blocks):
    B = xs[0].shape[0]
    us = [_unfold_tok(xs[i], _SHAPES[i][1], _SHAPES[i][2], _KS[i])
          for i in range(3)]
    tmp_skips = _select_tokens_all(us, _TOPK[:3]) + [xs[3]]
    folded = _fold_kv_multi([
        ([tmp_skips[j] for j in range(4) if j != idx],
         blocks[idx]['kvn_g'], blocks[idx]['kvn_b'], blocks[idx]['Wkv'],
         blocks[idx]['Wq'], blocks[idx]['Wout'], _CS[idx])
        for idx in range(3)])
    kts3, vts3 = _kv_project_all(tmp_skips[:3], blocks[3]['kvn_g'],
                                 blocks[3]['kvn_b'], blocks[3]['Wkv'],
                                 _HEADS * _CS[3])
    outs = []
    for idx in range(3):
        ap = blocks[idx]
        kk, vv = folded[idx]
        Ms = [tmp_skips[j].shape[1] for j in range(4) if j != idx]
        outs.append(_attention_fold(xs[idx], kk, vv, ap['qn_g'], ap['qn_b'],
                                    Ms, _CS[idx], _BN[idx]))
    ap = blocks[3]
    outs.append(_attention(xs[3], kts3, vts3, ap['qn_g'], ap['qn_b'],
                           ap['Wq'], ap['Wout'], _CS[3], _BN[3]))
    return outs


def kernel(x0, x1, x2, x3, params):
    B = x0.shape[0]
    xs = []
    for i, x in enumerate([x0, x1, x2, x3]):
        C, H, W = _SHAPES[i]
        xs.append(x.reshape(B, C, H * W).transpose(0, 2, 1))
    for t in range(_T_NUM):
        xs = _smla(xs, params[t])
    outs = []
    for i, o in enumerate(xs):
        C, H, W = _SHAPES[i]
        outs.append(o.transpose(0, 2, 1).reshape(B, C, H, W))
    return tuple(outs)
